# baseline (device time: 50927 ns/iter reference)
import os

import jax
import jax.numpy as jnp
from jax import lax
from jax.experimental import pallas as pl
from jax.experimental.pallas import tpu as pltpu

N_DEV = 4
S = 2
_SKIP_AG = bool(os.environ.get("SKIP_AG"))


def kernel(t, W):
    m_per, k = t.shape
    _, n = W.shape
    chunk = m_per // N_DEV
    sh = chunk // S
    nh = n // 2
    bf16 = jnp.bfloat16
    H = N_DEV - 1

    def body(t_ref, w_ref, out_ref,
             r1_send, r1_recv, r2_send, r2_recv, pre,
             r1_ssem, r1_rsem, r2_ssem, r2_rsem,
             ag_r_ssem, ag_r_rsem, ag_l_ssem, ag_l_rsem):
        my = lax.axis_index("i")
        left = lax.rem(my + N_DEV - 1, N_DEV)
        right = lax.rem(my + 1, N_DEV)

        def xor1(x):
            return x + 1 - 2 * lax.rem(x, 2)

        p1 = xor1(my)
        p2 = 3 - my

        subs = [
            dict(pa=p1, pb_pa=3 - p1, r2_peer=3 - my, kept1=3 - my),
            dict(pa=p2, pb_pa=xor1(p2), r2_peer=xor1(my), kept1=xor1(my)),
        ]

        def t_sub(idx, s):
            return t_ref[pl.ds(idx * chunk + s * sh, sh), :].astype(bf16)

        def copy(src, dst, ssem, rsem, dev):
            return pltpu.make_async_remote_copy(
                src_ref=src, dst_ref=dst, send_sem=ssem, recv_sem=rsem,
                device_id=(dev,), device_id_type=pl.DeviceIdType.MESH,
            )

        def out_rows(idx, s):
            return pl.ds(idx * chunk + s * sh, sh)

        def start_ag(s, h):
            sr_idx = lax.rem(my - h + 2 * N_DEV, N_DEV)
            sl_idx = lax.rem(my + h, N_DEV)
            r = copy(out_ref.at[out_rows(sr_idx, s), pl.ds(0, nh)],
                     out_ref.at[out_rows(sr_idx, s), pl.ds(0, nh)],
                     ag_r_ssem.at[s, h], ag_r_rsem.at[s, h], right)
            l = copy(out_ref.at[out_rows(sl_idx, s), pl.ds(nh, nh)],
                     out_ref.at[out_rows(sl_idx, s), pl.ds(nh, nh)],
                     ag_l_ssem.at[s, h], ag_l_rsem.at[s, h], left)
            r.start()
            l.start()
            return r, l

        for s, cfg in enumerate(subs):
            r1_send[s, 0, :, :] = t_sub(cfg["pa"], s)
            r1_send[s, 1, :, :] = t_sub(cfg["pb_pa"], s)

        barrier_sem = pltpu.get_barrier_semaphore()
        for nbr in (left, right):
            pl.semaphore_signal(
                barrier_sem, inc=1,
                device_id=(nbr,), device_id_type=pl.DeviceIdType.MESH,
            )
        pl.semaphore_wait(barrier_sem, 2)

        r1 = {}
        for s, cfg in enumerate(subs):
            for slot in range(2):
                c = copy(r1_send.at[s, slot], r1_recv.at[s, slot],
                         r1_ssem.at[s, slot], r1_rsem.at[s, slot],
                         cfg["pa"])
                c.start()
                r1[s, slot] = c

        for s, cfg in enumerate(subs):
            pre[s, 0, :, :] = t_sub(my, s)
            pre[s, 1, :, :] = t_sub(cfg["kept1"], s)
        w_bf = w_ref[:, :].astype(bf16)

        r2 = {}
        for s, cfg in enumerate(subs):
            r1[s, 1].wait()
            r2_send[s, :, :] = pre[s, 1] + r1_recv[s, 1]
            c = copy(r2_send.at[s], r2_recv.at[s],
                     r2_ssem.at[s], r2_rsem.at[s], cfg["r2_peer"])
            c.start()
            r2[s] = c

        ag = {}
        for s, cfg in enumerate(subs):
            r1[s, 0].wait()
            r2[s].wait()
            sum_full = pre[s, 0] + r1_recv[s, 0] + r2_recv[s]
            out_sub = jnp.dot(
                sum_full, w_bf, preferred_element_type=jnp.float32
            ).astype(bf16)
            out_ref[out_rows(my, s), :] = out_sub
            if not _SKIP_AG:
                ag[s, 0] = start_ag(s, 0)

        if not _SKIP_AG:
            for h in range(H - 1):
                for s in range(S):
                    r, l = ag[s, h]
                    r.wait()
                    l.wait()
                    ag[s, h + 1] = start_ag(s, h + 1)
            for s in range(S):
                r, l = ag[s, H - 1]
                r.wait()
                l.wait()

    return pl.pallas_call(
        body,
        out_shape=jax.ShapeDtypeStruct((m_per, n), bf16),
        in_specs=[
            pl.BlockSpec(memory_space=pltpu.VMEM),
            pl.BlockSpec(memory_space=pltpu.VMEM),
        ],
        out_specs=pl.BlockSpec(memory_space=pltpu.VMEM),
        scratch_shapes=[
            pltpu.VMEM((S, 2, sh, k), bf16),
            pltpu.VMEM((S, 2, sh, k), bf16),
            pltpu.VMEM((S, sh, k), bf16),
            pltpu.VMEM((S, sh, k), bf16),
            pltpu.VMEM((S, 2, sh, k), bf16),
            pltpu.SemaphoreType.DMA((S, 2)),
            pltpu.SemaphoreType.DMA((S, 2)),
            pltpu.SemaphoreType.DMA((S,)),
            pltpu.SemaphoreType.DMA((S,)),
            pltpu.SemaphoreType.DMA((S, H)),
            pltpu.SemaphoreType.DMA((S, H)),
            pltpu.SemaphoreType.DMA((S, H)),
            pltpu.SemaphoreType.DMA((S, H)),
        ],
        compiler_params=pltpu.CompilerParams(collective_id=0),
    )(t, W)


# device time: 46579 ns/iter; 1.0933x vs baseline; 1.0933x over previous
import os

import jax
import jax.numpy as jnp
from jax import lax
from jax.experimental import pallas as pl
from jax.experimental.pallas import tpu as pltpu

N_DEV = 4
S = 2
_SKIP_AG = bool(os.environ.get("SKIP_AG"))


def kernel(t, W):
    m_per, k = t.shape
    _, n = W.shape
    chunk = m_per // N_DEV
    sh = chunk // S
    kh = k // 2
    nh = n // 2
    bf16 = jnp.bfloat16
    H = N_DEV - 1

    def body(t_ref, w_ref, out_ref,
             rs_send, rs_recv, pre,
             rs_r_ssem, rs_r_rsem, rs_l_ssem, rs_l_rsem,
             ag_r_ssem, ag_r_rsem, ag_l_ssem, ag_l_rsem):
        my = lax.axis_index("i")
        left = lax.rem(my + N_DEV - 1, N_DEV)
        right = lax.rem(my + 1, N_DEV)

        def copy(src, dst, ssem, rsem, dev):
            return pltpu.make_async_remote_copy(
                src_ref=src, dst_ref=dst, send_sem=ssem, recv_sem=rsem,
                device_id=(dev,), device_id_type=pl.DeviceIdType.MESH,
            )

        def pre_idx(h):
            r_idx = lax.rem(my - h - 1 + 2 * N_DEV, N_DEV)
            l_idx = lax.rem(my + h + 3, N_DEV)
            return r_idx, l_idx

        def stage_pre(s, h):
            r_idx, l_idx = pre_idx(h)
            pre[s, h, :, :kh] = (
                t_ref[pl.ds(r_idx * chunk + s * sh, sh), :kh].astype(bf16))
            pre[s, h, :, kh:] = (
                t_ref[pl.ds(l_idx * chunk + s * sh, sh), kh:].astype(bf16))

        def start_rs(s, h):
            r = copy(rs_send.at[s, h, :, pl.ds(0, kh)],
                     rs_recv.at[s, h, :, pl.ds(0, kh)],
                     rs_r_ssem.at[s, h], rs_r_rsem.at[s, h], right)
            l = copy(rs_send.at[s, h, :, pl.ds(kh, kh)],
                     rs_recv.at[s, h, :, pl.ds(kh, kh)],
                     rs_l_ssem.at[s, h], rs_l_rsem.at[s, h], left)
            r.start()
            l.start()
            return r, l

        def out_rows(idx, s):
            return pl.ds(idx * chunk + s * sh, sh)

        def start_ag_r(s, h):
            idx = lax.rem(my + 1 - h + 2 * N_DEV, N_DEV)
            r = copy(out_ref.at[out_rows(idx, s), pl.ds(0, nh)],
                     out_ref.at[out_rows(idx, s), pl.ds(0, nh)],
                     ag_r_ssem.at[s, h], ag_r_rsem.at[s, h], right)
            r.start()
            return r

        def start_ag_l(s, h):
            idx = lax.rem(my + 1 + h, N_DEV)
            l = copy(out_ref.at[out_rows(idx, s), pl.ds(nh, nh)],
                     out_ref.at[out_rows(idx, s), pl.ds(nh, nh)],
                     ag_l_ssem.at[s, h], ag_l_rsem.at[s, h], left)
            l.start()
            return l

        for s in range(S):
            rs_send[s, 0, :, :kh] = (
                t_ref[pl.ds(my * chunk + s * sh, sh), :kh].astype(bf16))
            l0 = lax.rem(my + 2, N_DEV)
            rs_send[s, 0, :, kh:] = (
                t_ref[pl.ds(l0 * chunk + s * sh, sh), kh:].astype(bf16))

        barrier_sem = pltpu.get_barrier_semaphore()
        for nbr in (left, right):
            pl.semaphore_signal(
                barrier_sem, inc=1,
                device_id=(nbr,), device_id_type=pl.DeviceIdType.MESH,
            )
        pl.semaphore_wait(barrier_sem, 2)

        rs = {}
        ag = {}
        for s in range(S):
            rs[s, 0] = start_rs(s, 0)

        for s in range(S):
            stage_pre(s, 0)
        w_bf = w_ref[:, :].astype(bf16)

        for h in range(H - 1):
            for s in range(S):
                r, l = rs[s, h]
                r.wait()
                l.wait()
                rs_send[s, h + 1, :, :] = rs_recv[s, h] + pre[s, h]
                rs[s, h + 1] = start_rs(s, h + 1)
            for s in range(S):
                stage_pre(s, h + 1)

        my_out = lax.rem(my + 1, N_DEV)
        for s in range(S):
            r, l = rs[s, H - 1]
            r.wait()
            l.wait()
            sum_full = rs_recv[s, H - 1] + pre[s, H - 1]
            out_sub = jnp.dot(
                sum_full, w_bf, preferred_element_type=jnp.float32
            ).astype(bf16)
            out_ref[out_rows(my_out, s), :] = out_sub
            if not _SKIP_AG:
                ag[s, 0] = (start_ag_r(s, 0), start_ag_l(s, 0))

        if not _SKIP_AG:
            for h in range(H - 1):
                for s in range(S):
                    r, l = ag[s, h]
                    r.wait()
                    nr = start_ag_r(s, h + 1)
                    l.wait()
                    nl = start_ag_l(s, h + 1)
                    ag[s, h + 1] = (nr, nl)
            for s in range(S):
                r, l = ag[s, H - 1]
                r.wait()
                l.wait()

    return pl.pallas_call(
        body,
        out_shape=jax.ShapeDtypeStruct((m_per, n), bf16),
        in_specs=[
            pl.BlockSpec(memory_space=pltpu.VMEM),
            pl.BlockSpec(memory_space=pltpu.VMEM),
        ],
        out_specs=pl.BlockSpec(memory_space=pltpu.VMEM),
        scratch_shapes=[
            pltpu.VMEM((S, H, sh, k), bf16),
            pltpu.VMEM((S, H, sh, k), bf16),
            pltpu.VMEM((S, H, sh, k), bf16),
            pltpu.SemaphoreType.DMA((S, H)),
            pltpu.SemaphoreType.DMA((S, H)),
            pltpu.SemaphoreType.DMA((S, H)),
            pltpu.SemaphoreType.DMA((S, H)),
            pltpu.SemaphoreType.DMA((S, H)),
            pltpu.SemaphoreType.DMA((S, H)),
            pltpu.SemaphoreType.DMA((S, H)),
            pltpu.SemaphoreType.DMA((S, H)),
        ],
        compiler_params=pltpu.CompilerParams(collective_id=0),
    )(t, W)
